# Initial kernel scaffold; baseline (speedup 1.0000x reference)
import functools

import jax
import jax.numpy as jnp
from jax import lax
from jax.experimental import pallas as pl
from jax.experimental.pallas import tpu as pltpu
from jax.experimental.pallas import tpu_sc as plsc

N = 10000
E = 160000
D_IN = 256
D_H = 256
D_OUT = 64

NC = 2
NS = 16
NP = 12800
STRIPE = NP // NS
ROWB = 400
GRID = N // ROWB

_MESH = plsc.VectorSubcoreMesh(core_axis_name="c", subcore_axis_name="s")


def _fill_zero_rows(ref, nrows, ncols):
    z = jnp.zeros((16,), jnp.float32)

    def body(r, carry):
        for k in range(ncols // 16):
            ref[r, pl.ds(16 * k, 16)] = z
        return carry

    lax.fori_loop(0, nrows, body, 0)


_BD = 40
_EPW = E // (NC * NS)
_NCH_D = _EPW // _BD


def _deg_body(dst_hbm, out_hbm, idx_v, ones_v, zvec_v, deg_sh):
    c = lax.axis_index("c")
    s = lax.axis_index("s")
    one = jnp.ones((16,), jnp.float32)
    zero = jnp.zeros((16,), jnp.float32)
    for k in range(_BD // 16):
        ones_v[pl.ds(16 * k, 16)] = one
    for k in range(80 // 16):
        zvec_v[pl.ds(16 * k, 16)] = zero
    base_row = s * STRIPE
    for m in range(STRIPE // 80):
        pltpu.sync_copy(zvec_v, deg_sh.at[pl.ds(base_row + m * 80, 80)])
    plsc.subcore_barrier()
    base_e = (c * NS + s) * _EPW

    def chunk(j, carry):
        off = pl.multiple_of(base_e + j * _BD, 8)
        pltpu.sync_copy(dst_hbm.at[pl.ds(off, _BD)], idx_v)
        pltpu.sync_copy(ones_v, deg_sh.at[idx_v], add=True)
        return carry

    lax.fori_loop(0, _NCH_D, chunk, 0)
    plsc.subcore_barrier()
    pltpu.sync_copy(deg_sh.at[pl.ds(base_row, STRIPE)],
                    out_hbm.at[pl.ds(c * NP + base_row, STRIPE)])


_sc_deg = pl.kernel(
    _deg_body,
    out_type=jax.ShapeDtypeStruct((2 * NP,), jnp.float32),
    mesh=_MESH,
    scratch_types=[
        pltpu.VMEM((_BD,), jnp.int32),
        pltpu.VMEM((_BD,), jnp.float32),
        pltpu.VMEM((80,), jnp.float32),
        pltpu.VMEM_SHARED((NP,), jnp.float32),
    ],
)


_BC = 80
_EPT = E // NS
_NCH_C = _EPT // _BC


def _prop_col_body(table_hbm, src_hbm, dst_hbm, out_hbm,
                   src_v, dst_v, rows_v, zrow_v, agg_sh, sem):
    c = lax.axis_index("c")
    s = lax.axis_index("s")
    _fill_zero_rows(zrow_v, 80, 128)
    base_row = s * STRIPE
    for m in range(STRIPE // 80):
        pltpu.sync_copy(zrow_v, agg_sh.at[pl.ds(base_row + m * 80, 80)])
    plsc.subcore_barrier()
    base_src = c * E + s * _EPT
    base_dst = s * _EPT

    def chunk(j, carry):
        off = j * _BC
        pltpu.sync_copy(src_hbm.at[pl.ds(pl.multiple_of(base_src + off, 8), _BC)], src_v)
        pltpu.sync_copy(dst_hbm.at[pl.ds(pl.multiple_of(base_dst + off, 8), _BC)], dst_v)
        pltpu.async_copy(table_hbm.at[src_v], rows_v, sem).wait()
        pltpu.sync_copy(rows_v, agg_sh.at[dst_v], add=True)
        return carry

    lax.fori_loop(0, _NCH_C, chunk, 0)
    plsc.subcore_barrier()
    pltpu.sync_copy(agg_sh.at[pl.ds(base_row, STRIPE)],
                    out_hbm.at[pl.ds(c * NP + base_row, STRIPE)])


_sc_prop_col = pl.kernel(
    _prop_col_body,
    out_type=jax.ShapeDtypeStruct((2 * NP, 128), jnp.float32),
    mesh=_MESH,
    scratch_types=[
        pltpu.VMEM((_BC,), jnp.int32),
        pltpu.VMEM((_BC,), jnp.int32),
        pltpu.VMEM((_BC, 128), jnp.float32),
        pltpu.VMEM((80, 128), jnp.float32),
        pltpu.VMEM_SHARED((NP, 128), jnp.float32),
        pltpu.SemaphoreType.DMA,
    ],
)


_BE = 40
_NCH_E = _EPW // _BE


def _prop_edge_body(table_hbm, src_hbm, dst_hbm, out_hbm,
                    src_v, dst_v, rows_v, zrow_v, agg_sh, sem):
    c = lax.axis_index("c")
    s = lax.axis_index("s")
    _fill_zero_rows(zrow_v, 80, 64)
    base_row = s * STRIPE
    for m in range(STRIPE // 80):
        pltpu.sync_copy(zrow_v, agg_sh.at[pl.ds(base_row + m * 80, 80)])
    plsc.subcore_barrier()
    base_e = (c * NS + s) * _EPW

    def chunk(j, carry):
        off = pl.multiple_of(base_e + j * _BE, 8)
        pltpu.sync_copy(src_hbm.at[pl.ds(off, _BE)], src_v)
        pltpu.sync_copy(dst_hbm.at[pl.ds(off, _BE)], dst_v)
        pltpu.async_copy(table_hbm.at[src_v], rows_v, sem).wait()
        pltpu.sync_copy(rows_v, agg_sh.at[dst_v], add=True)
        return carry

    lax.fori_loop(0, _NCH_E, chunk, 0)
    plsc.subcore_barrier()
    pltpu.sync_copy(agg_sh.at[pl.ds(base_row, STRIPE)],
                    out_hbm.at[pl.ds(c * NP + base_row, STRIPE)])


_sc_prop_edge = pl.kernel(
    _prop_edge_body,
    out_type=jax.ShapeDtypeStruct((2 * NP, 64), jnp.float32),
    mesh=_MESH,
    scratch_types=[
        pltpu.VMEM((_BE,), jnp.int32),
        pltpu.VMEM((_BE,), jnp.int32),
        pltpu.VMEM((_BE, 64), jnp.float32),
        pltpu.VMEM((80, 64), jnp.float32),
        pltpu.VMEM_SHARED((NP, 64), jnp.float32),
        pltpu.SemaphoreType.DMA,
    ],
)


def _dis_from(degp_blk):
    deg = degp_blk[0] + degp_blk[1]
    return jnp.where(deg > 0, lax.rsqrt(jnp.maximum(deg, 1.0)), 0.0)


def _tc_in_body(x_ref, w_ref, degp_ref, out_ref):
    dis = _dis_from(degp_ref)
    h = jnp.dot(x_ref[...], w_ref[...], preferred_element_type=jnp.float32)
    hs = h * dis[:, None]
    out_ref[0] = hs[:, :128]
    out_ref[1] = hs[:, 128:]


_tc_in = pl.pallas_call(
    _tc_in_body,
    grid=(GRID,),
    in_specs=[
        pl.BlockSpec((ROWB, D_IN), lambda i: (i, 0)),
        pl.BlockSpec((D_IN, D_H), lambda i: (0, 0)),
        pl.BlockSpec((2, ROWB), lambda i: (0, i)),
    ],
    out_specs=pl.BlockSpec((2, ROWB, 128), lambda i: (0, i, 0)),
    out_shape=jax.ShapeDtypeStruct((2, NP, 128), jnp.float32),
)


def _tc_mid_body(agg_ref, degp_ref, b_ref, w_ref, out_ref):
    dis = _dis_from(degp_ref)
    agg = jnp.concatenate([agg_ref[0], agg_ref[1]], axis=-1)
    t = jnp.maximum(agg * dis[:, None] + b_ref[0], 0.0)
    h = jnp.dot(t, w_ref[...], preferred_element_type=jnp.float32)
    hs = h * dis[:, None]
    out_ref[0] = hs[:, :128]
    out_ref[1] = hs[:, 128:]


_tc_mid = pl.pallas_call(
    _tc_mid_body,
    grid=(GRID,),
    in_specs=[
        pl.BlockSpec((2, ROWB, 128), lambda i: (0, i, 0)),
        pl.BlockSpec((2, ROWB), lambda i: (0, i)),
        pl.BlockSpec((1, D_H), lambda i: (0, 0)),
        pl.BlockSpec((D_H, D_H), lambda i: (0, 0)),
    ],
    out_specs=pl.BlockSpec((2, ROWB, 128), lambda i: (0, i, 0)),
    out_shape=jax.ShapeDtypeStruct((2, NP, 128), jnp.float32),
)


def _tc_out_body(agg_ref, degp_ref, b_ref, w_ref, out_ref):
    dis = _dis_from(degp_ref)
    agg = jnp.concatenate([agg_ref[0], agg_ref[1]], axis=-1)
    t = jnp.maximum(agg * dis[:, None] + b_ref[0], 0.0)
    h = jnp.dot(t, w_ref[...], preferred_element_type=jnp.float32)
    out_ref[...] = h * dis[:, None]


_tc_out = pl.pallas_call(
    _tc_out_body,
    grid=(GRID,),
    in_specs=[
        pl.BlockSpec((2, ROWB, 128), lambda i: (0, i, 0)),
        pl.BlockSpec((2, ROWB), lambda i: (0, i)),
        pl.BlockSpec((1, D_H), lambda i: (0, 0)),
        pl.BlockSpec((D_H, D_OUT), lambda i: (0, 0)),
    ],
    out_specs=pl.BlockSpec((ROWB, D_OUT), lambda i: (i, 0)),
    out_shape=jax.ShapeDtypeStruct((NP, D_OUT), jnp.float32),
)


def _tc_final_body(aggp_ref, degp_ref, b_ref, out_ref):
    dis = _dis_from(degp_ref)
    p = aggp_ref[0] + aggp_ref[1]
    out_ref[...] = p * dis[:, None] + b_ref[0]


_tc_final = pl.pallas_call(
    _tc_final_body,
    grid=(GRID,),
    in_specs=[
        pl.BlockSpec((2, ROWB, D_OUT), lambda i: (0, i, 0)),
        pl.BlockSpec((2, ROWB), lambda i: (0, i)),
        pl.BlockSpec((1, D_OUT), lambda i: (0, 0)),
    ],
    out_specs=pl.BlockSpec((ROWB, D_OUT), lambda i: (i, 0)),
    out_shape=jax.ShapeDtypeStruct((N, D_OUT), jnp.float32),
)


def kernel(features, edge_index, W1, b1, W2, b2, W3, b3):
    src = edge_index[0].astype(jnp.int32)
    dst = edge_index[1].astype(jnp.int32)
    src_off = jnp.concatenate([src, src + NP])

    degp = _sc_deg(dst).reshape(2, NP)

    hs1 = _tc_in(features, W1, degp)
    agg1 = _sc_prop_col(hs1.reshape(2 * NP, 128), src_off, dst)
    hs2 = _tc_mid(agg1.reshape(2, NP, 128), degp, b1.reshape(1, D_H), W2)
    agg2 = _sc_prop_col(hs2.reshape(2 * NP, 128), src_off, dst)
    hs3 = _tc_out(agg2.reshape(2, NP, 128), degp, b2.reshape(1, D_H), W3)
    agg3 = _sc_prop_edge(hs3, src, dst)
    out = _tc_final(agg3.reshape(2, NP, 64), degp, b3.reshape(1, D_OUT))
    return out


# speedup vs baseline: 3.7315x; 3.7315x over previous
import functools

import jax
import jax.numpy as jnp
from jax import lax
from jax.experimental import pallas as pl
from jax.experimental.pallas import tpu as pltpu
from jax.experimental.pallas import tpu_sc as plsc

N = 10000
E = 160000
D_IN = 256
D_H = 256
D_OUT = 64

NC = 2
NS = 16
NP = 12800
STRIPE = NP // NS
ROWB = 400
GRID = N // ROWB

_MESH = plsc.VectorSubcoreMesh(core_axis_name="c", subcore_axis_name="s")


def _fill_zero_rows(ref, nrows, ncols):
    z = jnp.zeros((16,), jnp.float32)

    def body(r, carry):
        for k in range(ncols // 16):
            ref[r, pl.ds(16 * k, 16)] = z
        return carry

    lax.fori_loop(0, nrows, body, 0)


_BD = 40
_EPW = E // (NC * NS)
_NCH_D = _EPW // _BD


def _deg_body(dst_hbm, out_hbm, dst_v, rows_v, agg_sh):
    c = lax.axis_index("c")
    s = lax.axis_index("s")
    _fill_zero_rows(rows_v, _BD, 128)
    base_row = s * STRIPE
    for m in range(STRIPE // _BD):
        pltpu.sync_copy(rows_v, agg_sh.at[pl.ds(base_row + m * _BD, _BD)])
    plsc.subcore_barrier()
    one = jnp.ones((16,), jnp.float32)

    def ofill(r, carry):
        for k in range(128 // 16):
            rows_v[r, pl.ds(16 * k, 16)] = one
        return carry

    lax.fori_loop(0, _BD, ofill, 0)
    base_e = (c * NS + s) * _EPW

    def chunk(j, carry):
        off = pl.multiple_of(base_e + j * _BD, 8)
        pltpu.sync_copy(dst_hbm.at[pl.ds(off, _BD)], dst_v)
        pltpu.sync_copy(rows_v, agg_sh.at[dst_v], add=True)
        return carry

    lax.fori_loop(0, _NCH_D, chunk, 0)
    plsc.subcore_barrier()
    for m in range(STRIPE // _BD):
        pltpu.sync_copy(agg_sh.at[pl.ds(base_row + m * _BD, _BD)], rows_v)
        pltpu.sync_copy(rows_v, out_hbm.at[pl.ds(c * NP + base_row + m * _BD, _BD)])


_sc_deg = pl.kernel(
    _deg_body,
    out_type=jax.ShapeDtypeStruct((2 * NP, 128), jnp.float32),
    mesh=_MESH,
    scratch_types=[
        pltpu.VMEM((_BD,), jnp.int32),
        pltpu.VMEM((_BD, 128), jnp.float32),
        pltpu.VMEM_SHARED((NP, 128), jnp.float32),
    ],
)


_BC = 80
_EPT = E // NS
_NCH_C = _EPT // _BC


def _prop_col_body(table_hbm, src_hbm, dst_hbm, out_hbm,
                   src_v, dst_v, rows_v, zrow_v, agg_sh, sem):
    c = lax.axis_index("c")
    s = lax.axis_index("s")
    _fill_zero_rows(zrow_v, 80, 128)
    base_row = s * STRIPE
    for m in range(STRIPE // 80):
        pltpu.sync_copy(zrow_v, agg_sh.at[pl.ds(base_row + m * 80, 80)])
    plsc.subcore_barrier()
    base_src = c * E + s * _EPT
    base_dst = s * _EPT

    def chunk(j, carry):
        off = j * _BC
        pltpu.sync_copy(src_hbm.at[pl.ds(pl.multiple_of(base_src + off, 8), _BC)], src_v)
        pltpu.sync_copy(dst_hbm.at[pl.ds(pl.multiple_of(base_dst + off, 8), _BC)], dst_v)
        pltpu.async_copy(table_hbm.at[src_v], rows_v, sem).wait()
        pltpu.sync_copy(rows_v, agg_sh.at[dst_v], add=True)
        return carry

    lax.fori_loop(0, _NCH_C, chunk, 0)
    plsc.subcore_barrier()
    for m in range(STRIPE // 80):
        pltpu.sync_copy(agg_sh.at[pl.ds(base_row + m * 80, 80)], zrow_v)
        pltpu.sync_copy(zrow_v, out_hbm.at[pl.ds(c * NP + base_row + m * 80, 80)])


_sc_prop_col = pl.kernel(
    _prop_col_body,
    out_type=jax.ShapeDtypeStruct((2 * NP, 128), jnp.float32),
    mesh=_MESH,
    scratch_types=[
        pltpu.VMEM((_BC,), jnp.int32),
        pltpu.VMEM((_BC,), jnp.int32),
        pltpu.VMEM((_BC, 128), jnp.float32),
        pltpu.VMEM((80, 128), jnp.float32),
        pltpu.VMEM_SHARED((NP, 128), jnp.float32),
        pltpu.SemaphoreType.DMA,
    ],
)


_BE = 40
_NCH_E = _EPW // _BE


def _prop_edge_body(table_hbm, src_hbm, dst_hbm, out_hbm,
                    src_v, dst_v, rows_v, zrow_v, agg_sh, sem):
    c = lax.axis_index("c")
    s = lax.axis_index("s")
    _fill_zero_rows(zrow_v, 80, 128)
    base_row = s * STRIPE
    for m in range(STRIPE // 80):
        pltpu.sync_copy(zrow_v, agg_sh.at[pl.ds(base_row + m * 80, 80)])
    plsc.subcore_barrier()
    base_e = (c * NS + s) * _EPW

    def chunk(j, carry):
        off = pl.multiple_of(base_e + j * _BE, 8)
        pltpu.sync_copy(src_hbm.at[pl.ds(off, _BE)], src_v)
        pltpu.sync_copy(dst_hbm.at[pl.ds(off, _BE)], dst_v)
        pltpu.async_copy(table_hbm.at[src_v], rows_v, sem).wait()
        pltpu.sync_copy(rows_v, agg_sh.at[dst_v], add=True)
        return carry

    lax.fori_loop(0, _NCH_E, chunk, 0)
    plsc.subcore_barrier()
    for m in range(STRIPE // 80):
        pltpu.sync_copy(agg_sh.at[pl.ds(base_row + m * 80, 80)], zrow_v)
        pltpu.sync_copy(zrow_v, out_hbm.at[pl.ds(c * NP + base_row + m * 80, 80)])


_sc_prop_edge = pl.kernel(
    _prop_edge_body,
    out_type=jax.ShapeDtypeStruct((2 * NP, 128), jnp.float32),
    mesh=_MESH,
    scratch_types=[
        pltpu.VMEM((_BE,), jnp.int32),
        pltpu.VMEM((_BE,), jnp.int32),
        pltpu.VMEM((_BE, 128), jnp.float32),
        pltpu.VMEM((80, 128), jnp.float32),
        pltpu.VMEM_SHARED((NP, 128), jnp.float32),
        pltpu.SemaphoreType.DMA,
    ],
)


def _dis_from(d0_ref, d1_ref):
    deg = d0_ref[...] + d1_ref[...]
    return jnp.where(deg > 0, lax.rsqrt(jnp.maximum(deg, 1.0)), 0.0)


def _tc_in_body(x_ref, w_ref, d0_ref, d1_ref, out_ref):
    dis = _dis_from(d0_ref, d1_ref)
    h = jnp.dot(x_ref[...], w_ref[...], preferred_element_type=jnp.float32)
    hs = h * dis
    out_ref[0] = hs[:, :128]
    out_ref[1] = hs[:, 128:]


_tc_in = pl.pallas_call(
    _tc_in_body,
    grid=(GRID,),
    in_specs=[
        pl.BlockSpec((ROWB, D_IN), lambda i: (i, 0)),
        pl.BlockSpec((D_IN, D_H), lambda i: (0, 0)),
        pl.BlockSpec((ROWB, 1), lambda i: (i, 0)),
        pl.BlockSpec((ROWB, 1), lambda i: (i, 0)),
    ],
    out_specs=pl.BlockSpec((2, ROWB, 128), lambda i: (0, i, 0)),
    out_shape=jax.ShapeDtypeStruct((2, NP, 128), jnp.float32),
)


def _tc_mid_body(agg_ref, d0_ref, d1_ref, b_ref, w_ref, out_ref):
    dis = _dis_from(d0_ref, d1_ref)
    agg = jnp.concatenate([agg_ref[0], agg_ref[1]], axis=-1)
    t = jnp.maximum(agg * dis + b_ref[0], 0.0)
    h = jnp.dot(t, w_ref[...], preferred_element_type=jnp.float32)
    hs = h * dis
    out_ref[0] = hs[:, :128]
    out_ref[1] = hs[:, 128:]


_tc_mid = pl.pallas_call(
    _tc_mid_body,
    grid=(GRID,),
    in_specs=[
        pl.BlockSpec((2, ROWB, 128), lambda i: (0, i, 0)),
        pl.BlockSpec((ROWB, 1), lambda i: (i, 0)),
        pl.BlockSpec((ROWB, 1), lambda i: (i, 0)),
        pl.BlockSpec((1, D_H), lambda i: (0, 0)),
        pl.BlockSpec((D_H, D_H), lambda i: (0, 0)),
    ],
    out_specs=pl.BlockSpec((2, ROWB, 128), lambda i: (0, i, 0)),
    out_shape=jax.ShapeDtypeStruct((2, NP, 128), jnp.float32),
)


def _tc_out_body(agg_ref, d0_ref, d1_ref, b_ref, w_ref, out_ref):
    dis = _dis_from(d0_ref, d1_ref)
    agg = jnp.concatenate([agg_ref[0], agg_ref[1]], axis=-1)
    t = jnp.maximum(agg * dis + b_ref[0], 0.0)
    h = jnp.dot(t, w_ref[...], preferred_element_type=jnp.float32)
    hs = h * dis
    out_ref[...] = jnp.concatenate(
        [hs, jnp.zeros((ROWB, 128 - D_OUT), jnp.float32)], axis=-1)


_tc_out = pl.pallas_call(
    _tc_out_body,
    grid=(GRID,),
    in_specs=[
        pl.BlockSpec((2, ROWB, 128), lambda i: (0, i, 0)),
        pl.BlockSpec((ROWB, 1), lambda i: (i, 0)),
        pl.BlockSpec((ROWB, 1), lambda i: (i, 0)),
        pl.BlockSpec((1, D_H), lambda i: (0, 0)),
        pl.BlockSpec((D_H, D_OUT), lambda i: (0, 0)),
    ],
    out_specs=pl.BlockSpec((ROWB, 128), lambda i: (i, 0)),
    out_shape=jax.ShapeDtypeStruct((NP, 128), jnp.float32),
)


def _tc_final_body(aggp_ref, d0_ref, d1_ref, b_ref, out_ref):
    dis = _dis_from(d0_ref, d1_ref)
    p = aggp_ref[0, :, :D_OUT] + aggp_ref[1, :, :D_OUT]
    out_ref[...] = p * dis + b_ref[0]


_tc_final = pl.pallas_call(
    _tc_final_body,
    grid=(GRID,),
    in_specs=[
        pl.BlockSpec((2, ROWB, 128), lambda i: (0, i, 0)),
        pl.BlockSpec((ROWB, 1), lambda i: (i, 0)),
        pl.BlockSpec((ROWB, 1), lambda i: (i, 0)),
        pl.BlockSpec((1, D_OUT), lambda i: (0, 0)),
    ],
    out_specs=pl.BlockSpec((ROWB, D_OUT), lambda i: (i, 0)),
    out_shape=jax.ShapeDtypeStruct((N, D_OUT), jnp.float32),
)


def kernel(features, edge_index, W1, b1, W2, b2, W3, b3):
    src = edge_index[0].astype(jnp.int32)
    dst = edge_index[1].astype(jnp.int32)
    src_off = jnp.concatenate([src, src + NP])

    degp = _sc_deg(dst)
    d0 = degp[:NP, :1]
    d1 = degp[NP:, :1]

    hs1 = _tc_in(features, W1, d0, d1)
    agg1 = _sc_prop_col(hs1.reshape(2 * NP, 128), src_off, dst)
    hs2 = _tc_mid(agg1.reshape(2, NP, 128), d0, d1, b1.reshape(1, D_H), W2)
    agg2 = _sc_prop_col(hs2.reshape(2 * NP, 128), src_off, dst)
    hs3 = _tc_out(agg2.reshape(2, NP, 128), d0, d1, b2.reshape(1, D_H), W3)
    agg3 = _sc_prop_edge(hs3, src, dst)
    out = _tc_final(agg3.reshape(2, NP, 128), d0, d1, b3.reshape(1, D_OUT))
    return out


# speedup vs baseline: 4.4312x; 1.1875x over previous
import jax
import jax.numpy as jnp
from jax import lax
from jax.experimental import pallas as pl
from jax.experimental.pallas import tpu as pltpu
from jax.experimental.pallas import tpu_sc as plsc

N = 10000
E = 160000
D_IN = 256
D_H = 256
D_OUT = 64

NC = 2
NS = 16
NP = 12800
STRIPE = NP // NS
ROWB = 400
GRID = N // ROWB

ACC = 10240
ACC_STRIPE = ACC // NS
B = 64
EPT = E // NS
EPT_PAD = 10240
NCH_C = EPT_PAD // B
EPW = E // (NC * NS)
EPW_PAD = 5120
NCH_E = EPW_PAD // B

_MESH = plsc.VectorSubcoreMesh(core_axis_name="c", subcore_axis_name="s")


def _fill_const_rows(ref, nrows, ncols, value):
    v = jnp.full((16,), value, jnp.float32)

    def body(r, carry):
        for k in range(ncols // 16):
            ref[r, pl.ds(16 * k, 16)] = v
        return carry

    lax.fori_loop(0, nrows, body, 0)


def _zero_stripe(stage, agg_sh, base_row):
    _fill_const_rows(stage, B, 128, 0.0)
    for m in range(ACC_STRIPE // B):
        pltpu.sync_copy(stage, agg_sh.at[pl.ds(base_row + m * B, B)])


def _copy_out_stripe(stage, agg_sh, out_hbm, base_row, out_base):
    for m in range(ACC_STRIPE // B):
        pltpu.sync_copy(agg_sh.at[pl.ds(base_row + m * B, B)], stage)
        pltpu.sync_copy(stage, out_hbm.at[pl.ds(out_base + m * B, B)])


def _gather_scatter_phase(table_hbm, src_v, dst_v, rows0, rows1, agg_sh,
                          sem0, sem1, nch):
    rows = (rows0, rows1)
    sems = (sem0, sem1)
    pltpu.async_copy(table_hbm.at[src_v.at[0]], rows0, sem0)
    pltpu.async_copy(table_hbm.at[src_v.at[1]], rows1, sem1)

    def pair(i, carry):
        for b in range(2):
            j = 2 * i + b
            pltpu.make_async_copy(table_hbm.at[src_v.at[j]], rows[b],
                                  sems[b]).wait()
            pltpu.sync_copy(rows[b], agg_sh.at[dst_v.at[j]], add=True)
            pltpu.async_copy(table_hbm.at[src_v.at[j + 2]], rows[b], sems[b])
        return carry

    lax.fori_loop(0, nch // 2 - 1, pair, 0)
    for b in range(2):
        j = nch - 2 + b
        pltpu.make_async_copy(table_hbm.at[src_v.at[j]], rows[b],
                              sems[b]).wait()
        pltpu.sync_copy(rows[b], agg_sh.at[dst_v.at[j]], add=True)


def _deg_body(dst_hbm, out_hbm, dst_v, ones_v, agg_sh, sem):
    c = lax.axis_index("c")
    s = lax.axis_index("s")
    base_row = s * ACC_STRIPE
    _zero_stripe(ones_v, agg_sh, base_row)
    plsc.subcore_barrier()
    _fill_const_rows(ones_v, B, 128, 1.0)
    w = c * NS + s
    pltpu.sync_copy(dst_hbm.at[pl.ds(w * NCH_E, NCH_E)], dst_v)

    def quad(i, carry):
        for b in range(4):
            j = 4 * i + b
            pltpu.async_copy(ones_v, agg_sh.at[dst_v.at[j]], sem, add=True)
        for b in range(4):
            j = 4 * i + b
            pltpu.make_async_copy(ones_v, agg_sh.at[dst_v.at[j]], sem).wait()
        return carry

    lax.fori_loop(0, NCH_E // 4, quad, 0)
    plsc.subcore_barrier()
    _copy_out_stripe(ones_v, agg_sh, out_hbm, base_row, c * NP + base_row)


_sc_deg = pl.kernel(
    _deg_body,
    out_type=jax.ShapeDtypeStruct((2 * NP, 128), jnp.float32),
    mesh=_MESH,
    scratch_types=[
        pltpu.VMEM((NCH_E, B), jnp.int32),
        pltpu.VMEM((B, 128), jnp.float32),
        pltpu.VMEM_SHARED((ACC, 128), jnp.float32),
        pltpu.SemaphoreType.DMA,
    ],
)


def _prop_col_body(table_hbm, srcI_hbm, dstI_hbm, out_hbm,
                   src_v, dst_v, rows0, rows1, agg_sh, sem0, sem1):
    c = lax.axis_index("c")
    s = lax.axis_index("s")
    base_row = s * ACC_STRIPE
    _zero_stripe(rows0, agg_sh, base_row)
    plsc.subcore_barrier()
    off = c * NP
    h = NCH_C // 2
    for phase in range(2):
        pltpu.sync_copy(srcI_hbm.at[pl.ds(s * NCH_C + phase * h, h)], src_v)
        pltpu.sync_copy(dstI_hbm.at[pl.ds(s * NCH_C + phase * h, h)], dst_v)

        def addoff(r, carry):
            for k in range(B // 16):
                src_v[r, pl.ds(16 * k, 16)] = src_v[r, pl.ds(16 * k, 16)] + off
            return carry

        lax.fori_loop(0, h, addoff, 0)
        _gather_scatter_phase(table_hbm, src_v, dst_v, rows0, rows1, agg_sh,
                              sem0, sem1, h)
    plsc.subcore_barrier()
    _copy_out_stripe(rows0, agg_sh, out_hbm, base_row, c * NP + base_row)


_sc_prop_col = pl.kernel(
    _prop_col_body,
    out_type=jax.ShapeDtypeStruct((2 * NP, 128), jnp.float32),
    mesh=_MESH,
    scratch_types=[
        pltpu.VMEM((NCH_C // 2, B), jnp.int32),
        pltpu.VMEM((NCH_C // 2, B), jnp.int32),
        pltpu.VMEM((B, 128), jnp.float32),
        pltpu.VMEM((B, 128), jnp.float32),
        pltpu.VMEM_SHARED((ACC, 128), jnp.float32),
        pltpu.SemaphoreType.DMA,
        pltpu.SemaphoreType.DMA,
    ],
)


def _prop_edge_body(table_hbm, srcI_hbm, dstI_hbm, out_hbm,
                    src_v, dst_v, rows0, rows1, agg_sh, sem0, sem1):
    c = lax.axis_index("c")
    s = lax.axis_index("s")
    base_row = s * ACC_STRIPE
    _zero_stripe(rows0, agg_sh, base_row)
    plsc.subcore_barrier()
    w = c * NS + s
    pltpu.sync_copy(srcI_hbm.at[pl.ds(w * NCH_E, NCH_E)], src_v)
    pltpu.sync_copy(dstI_hbm.at[pl.ds(w * NCH_E, NCH_E)], dst_v)
    _gather_scatter_phase(table_hbm, src_v, dst_v, rows0, rows1, agg_sh,
                          sem0, sem1, NCH_E)
    plsc.subcore_barrier()
    _copy_out_stripe(rows0, agg_sh, out_hbm, base_row, c * NP + base_row)


_sc_prop_edge = pl.kernel(
    _prop_edge_body,
    out_type=jax.ShapeDtypeStruct((2 * NP, 128), jnp.float32),
    mesh=_MESH,
    scratch_types=[
        pltpu.VMEM((NCH_E, B), jnp.int32),
        pltpu.VMEM((NCH_E, B), jnp.int32),
        pltpu.VMEM((B, 128), jnp.float32),
        pltpu.VMEM((B, 128), jnp.float32),
        pltpu.VMEM_SHARED((ACC, 128), jnp.float32),
        pltpu.SemaphoreType.DMA,
        pltpu.SemaphoreType.DMA,
    ],
)


def _dis_from(d0_ref, d1_ref):
    deg = d0_ref[...] + d1_ref[...]
    return jnp.where(deg > 0, lax.rsqrt(jnp.maximum(deg, 1.0)), 0.0)


def _tc_in_body(x_ref, w_ref, d0_ref, d1_ref, out_ref):
    dis = _dis_from(d0_ref, d1_ref)
    h = jnp.dot(x_ref[...], w_ref[...], preferred_element_type=jnp.float32)
    hs = h * dis
    out_ref[0] = hs[:, :128]
    out_ref[1] = hs[:, 128:]


_tc_in = pl.pallas_call(
    _tc_in_body,
    grid=(GRID,),
    in_specs=[
        pl.BlockSpec((ROWB, D_IN), lambda i: (i, 0)),
        pl.BlockSpec((D_IN, D_H), lambda i: (0, 0)),
        pl.BlockSpec((ROWB, 1), lambda i: (i, 0)),
        pl.BlockSpec((ROWB, 1), lambda i: (i, 0)),
    ],
    out_specs=pl.BlockSpec((2, ROWB, 128), lambda i: (0, i, 0)),
    out_shape=jax.ShapeDtypeStruct((2, NP, 128), jnp.float32),
)


def _tc_mid_body(agg_ref, d0_ref, d1_ref, b_ref, w_ref, out_ref):
    dis = _dis_from(d0_ref, d1_ref)
    agg = jnp.concatenate([agg_ref[0], agg_ref[1]], axis=-1)
    t = jnp.maximum(agg * dis + b_ref[0], 0.0)
    h = jnp.dot(t, w_ref[...], preferred_element_type=jnp.float32)
    hs = h * dis
    out_ref[0] = hs[:, :128]
    out_ref[1] = hs[:, 128:]


_tc_mid = pl.pallas_call(
    _tc_mid_body,
    grid=(GRID,),
    in_specs=[
        pl.BlockSpec((2, ROWB, 128), lambda i: (0, i, 0)),
        pl.BlockSpec((ROWB, 1), lambda i: (i, 0)),
        pl.BlockSpec((ROWB, 1), lambda i: (i, 0)),
        pl.BlockSpec((1, D_H), lambda i: (0, 0)),
        pl.BlockSpec((D_H, D_H), lambda i: (0, 0)),
    ],
    out_specs=pl.BlockSpec((2, ROWB, 128), lambda i: (0, i, 0)),
    out_shape=jax.ShapeDtypeStruct((2, NP, 128), jnp.float32),
)


def _tc_out_body(agg_ref, d0_ref, d1_ref, b_ref, w_ref, out_ref):
    dis = _dis_from(d0_ref, d1_ref)
    agg = jnp.concatenate([agg_ref[0], agg_ref[1]], axis=-1)
    t = jnp.maximum(agg * dis + b_ref[0], 0.0)
    h = jnp.dot(t, w_ref[...], preferred_element_type=jnp.float32)
    hs = h * dis
    out_ref[...] = jnp.concatenate(
        [hs, jnp.zeros((ROWB, 128 - D_OUT), jnp.float32)], axis=-1)


_tc_out = pl.pallas_call(
    _tc_out_body,
    grid=(GRID,),
    in_specs=[
        pl.BlockSpec((2, ROWB, 128), lambda i: (0, i, 0)),
        pl.BlockSpec((ROWB, 1), lambda i: (i, 0)),
        pl.BlockSpec((ROWB, 1), lambda i: (i, 0)),
        pl.BlockSpec((1, D_H), lambda i: (0, 0)),
        pl.BlockSpec((D_H, D_OUT), lambda i: (0, 0)),
    ],
    out_specs=pl.BlockSpec((ROWB, 128), lambda i: (i, 0)),
    out_shape=jax.ShapeDtypeStruct((NP, 128), jnp.float32),
)


def _tc_final_body(aggp_ref, d0_ref, d1_ref, b_ref, out_ref):
    dis = _dis_from(d0_ref, d1_ref)
    p = aggp_ref[0, :, :D_OUT] + aggp_ref[1, :, :D_OUT]
    out_ref[...] = p * dis + b_ref[0]


_tc_final = pl.pallas_call(
    _tc_final_body,
    grid=(GRID,),
    in_specs=[
        pl.BlockSpec((2, ROWB, 128), lambda i: (0, i, 0)),
        pl.BlockSpec((ROWB, 1), lambda i: (i, 0)),
        pl.BlockSpec((ROWB, 1), lambda i: (i, 0)),
        pl.BlockSpec((1, D_OUT), lambda i: (0, 0)),
    ],
    out_specs=pl.BlockSpec((ROWB, D_OUT), lambda i: (i, 0)),
    out_shape=jax.ShapeDtypeStruct((N, D_OUT), jnp.float32),
)


def _pad_edges(x, per, pad_to, fill):
    groups = x.reshape(-1, per)
    padded = jnp.concatenate(
        [groups, jnp.broadcast_to(fill, (groups.shape[0], pad_to - per))],
        axis=1)
    return padded


def kernel(features, edge_index, W1, b1, W2, b2, W3, b3):
    src = edge_index[0].astype(jnp.int32)
    dst = edge_index[1].astype(jnp.int32)

    pad_c = N + (jnp.arange(EPT_PAD - EPT, dtype=jnp.int32) % (ACC - N))
    pad_e = N + (jnp.arange(EPW_PAD - EPW, dtype=jnp.int32) % (ACC - N))

    srcI_c = _pad_edges(src, EPT, EPT_PAD, jnp.int32(0)).reshape(-1, B)
    dstI_c = _pad_edges(dst, EPT, EPT_PAD, pad_c).reshape(-1, B)

    srcI_e = _pad_edges(src, EPW, EPW_PAD, jnp.int32(0)).reshape(-1, B)
    dstI_e = _pad_edges(dst, EPW, EPW_PAD, pad_e).reshape(-1, B)

    degp = _sc_deg(dstI_e)
    d0 = degp[:NP, :1]
    d1 = degp[NP:, :1]

    hs1 = _tc_in(features, W1, d0, d1)
    agg1 = _sc_prop_col(hs1.reshape(2 * NP, 128), srcI_c, dstI_c)
    hs2 = _tc_mid(agg1.reshape(2, NP, 128), d0, d1, b1.reshape(1, D_H), W2)
    agg2 = _sc_prop_col(hs2.reshape(2 * NP, 128), srcI_c, dstI_c)
    hs3 = _tc_out(agg2.reshape(2, NP, 128), d0, d1, b2.reshape(1, D_H), W3)
    agg3 = _sc_prop_edge(hs3, srcI_e, dstI_e)
    out = _tc_final(agg3.reshape(2, NP, 128), d0, d1, b3.reshape(1, D_OUT))
    return out


# speedup vs baseline: 4.5758x; 1.0326x over previous
import jax
import jax.numpy as jnp
from jax import lax
from jax.experimental import pallas as pl
from jax.experimental.pallas import tpu as pltpu
from jax.experimental.pallas import tpu_sc as plsc

N = 10000
E = 160000
D_IN = 256
D_H = 256
D_OUT = 64

NC = 2
NS = 16
NP = 12800
STRIPE = NP // NS
ROWB = 400
GRID = N // ROWB

ACC = 10240
ACC_STRIPE = ACC // NS
B = 64
EPT = E // NS
EPT_PAD = 10240
NCH_C = EPT_PAD // B
EPW = E // (NC * NS)
EPW_PAD = 5120
NCH_E = EPW_PAD // B

_MESH = plsc.VectorSubcoreMesh(core_axis_name="c", subcore_axis_name="s")


def _fill_const_rows(ref, nrows, ncols, value):
    v = jnp.full((16,), value, jnp.float32)

    def body(r, carry):
        for k in range(ncols // 16):
            ref[r, pl.ds(16 * k, 16)] = v
        return carry

    lax.fori_loop(0, nrows, body, 0)


def _zero_stripe(stage, agg_sh, base_row):
    _fill_const_rows(stage, B, 128, 0.0)
    for m in range(ACC_STRIPE // B):
        pltpu.sync_copy(stage, agg_sh.at[pl.ds(base_row + m * B, B)])


def _copy_out_stripe(stage, agg_sh, out_hbm, base_row, out_base):
    for m in range(ACC_STRIPE // B):
        pltpu.sync_copy(agg_sh.at[pl.ds(base_row + m * B, B)], stage)
        pltpu.sync_copy(stage, out_hbm.at[pl.ds(out_base + m * B, B)])


def _gather_scatter_phase(table_hbm, src_v, dst_v, rows, agg_sh,
                          sems_g, sems_s, nch):
    for b in range(4):
        pltpu.async_copy(table_hbm.at[src_v.at[b]], rows[b], sems_g[b])

    def quad(i, carry):
        for b in range(4):
            j = 4 * i + b
            pltpu.make_async_copy(table_hbm.at[src_v.at[j]], rows[b],
                                  sems_g[b]).wait()
            pltpu.async_copy(rows[b], agg_sh.at[dst_v.at[j]], sems_s[b],
                             add=True)
        for b in range(4):
            j = 4 * i + b
            pltpu.make_async_copy(rows[b], agg_sh.at[dst_v.at[j]],
                                  sems_s[b]).wait()
            pltpu.async_copy(table_hbm.at[src_v.at[j + 4]], rows[b],
                             sems_g[b])
        return carry

    lax.fori_loop(0, nch // 4 - 1, quad, 0)
    for b in range(4):
        j = nch - 4 + b
        pltpu.make_async_copy(table_hbm.at[src_v.at[j]], rows[b],
                              sems_g[b]).wait()
        pltpu.async_copy(rows[b], agg_sh.at[dst_v.at[j]], sems_s[b], add=True)
    for b in range(4):
        j = nch - 4 + b
        pltpu.make_async_copy(rows[b], agg_sh.at[dst_v.at[j]],
                              sems_s[b]).wait()


def _deg_body(dst_hbm, out_hbm, dst_v, ones_v, agg_sh, sem):
    c = lax.axis_index("c")
    s = lax.axis_index("s")
    base_row = s * ACC_STRIPE
    _zero_stripe(ones_v, agg_sh, base_row)
    plsc.subcore_barrier()
    _fill_const_rows(ones_v, B, 128, 1.0)
    w = c * NS + s
    pltpu.sync_copy(dst_hbm.at[pl.ds(w * NCH_E, NCH_E)], dst_v)

    def quad(i, carry):
        for b in range(4):
            j = 4 * i + b
            pltpu.async_copy(ones_v, agg_sh.at[dst_v.at[j]], sem, add=True)
        for b in range(4):
            j = 4 * i + b
            pltpu.make_async_copy(ones_v, agg_sh.at[dst_v.at[j]], sem).wait()
        return carry

    lax.fori_loop(0, NCH_E // 4, quad, 0)
    plsc.subcore_barrier()
    _copy_out_stripe(ones_v, agg_sh, out_hbm, base_row, c * NP + base_row)


_sc_deg = pl.kernel(
    _deg_body,
    out_type=jax.ShapeDtypeStruct((2 * NP, 128), jnp.float32),
    mesh=_MESH,
    scratch_types=[
        pltpu.VMEM((NCH_E, B), jnp.int32),
        pltpu.VMEM((B, 128), jnp.float32),
        pltpu.VMEM_SHARED((ACC, 128), jnp.float32),
        pltpu.SemaphoreType.DMA,
    ],
)


def _prop_col_body(table_hbm, srcI_hbm, dstI_hbm, out_hbm,
                   src_v, dst_v, rows0, rows1, rows2, rows3, agg_sh,
                   sg0, sg1, sg2, sg3, ss0, ss1, ss2, ss3):
    c = lax.axis_index("c")
    s = lax.axis_index("s")
    base_row = s * ACC_STRIPE
    _zero_stripe(rows0, agg_sh, base_row)
    plsc.subcore_barrier()
    off = c * NP
    h = NCH_C // 4
    for phase in range(4):
        pltpu.sync_copy(srcI_hbm.at[pl.ds(s * NCH_C + phase * h, h)], src_v)
        pltpu.sync_copy(dstI_hbm.at[pl.ds(s * NCH_C + phase * h, h)], dst_v)

        def addoff(r, carry):
            for k in range(B // 16):
                src_v[r, pl.ds(16 * k, 16)] = src_v[r, pl.ds(16 * k, 16)] + off
            return carry

        lax.fori_loop(0, h, addoff, 0)
        _gather_scatter_phase(table_hbm, src_v, dst_v,
                              (rows0, rows1, rows2, rows3), agg_sh,
                              (sg0, sg1, sg2, sg3), (ss0, ss1, ss2, ss3), h)
    plsc.subcore_barrier()
    _copy_out_stripe(rows0, agg_sh, out_hbm, base_row, c * NP + base_row)


_sc_prop_col = pl.kernel(
    _prop_col_body,
    out_type=jax.ShapeDtypeStruct((2 * NP, 128), jnp.float32),
    mesh=_MESH,
    scratch_types=[
        pltpu.VMEM((NCH_C // 4, B), jnp.int32),
        pltpu.VMEM((NCH_C // 4, B), jnp.int32),
        pltpu.VMEM((B, 128), jnp.float32),
        pltpu.VMEM((B, 128), jnp.float32),
        pltpu.VMEM((B, 128), jnp.float32),
        pltpu.VMEM((B, 128), jnp.float32),
        pltpu.VMEM_SHARED((ACC, 128), jnp.float32),
        pltpu.SemaphoreType.DMA,
        pltpu.SemaphoreType.DMA,
        pltpu.SemaphoreType.DMA,
        pltpu.SemaphoreType.DMA,
        pltpu.SemaphoreType.DMA,
        pltpu.SemaphoreType.DMA,
        pltpu.SemaphoreType.DMA,
        pltpu.SemaphoreType.DMA,
    ],
)


def _prop_edge_body(table_hbm, srcI_hbm, dstI_hbm, out_hbm,
                    src_v, dst_v, rows0, rows1, rows2, rows3, agg_sh,
                    sg0, sg1, sg2, sg3, ss0, ss1, ss2, ss3):
    c = lax.axis_index("c")
    s = lax.axis_index("s")
    base_row = s * ACC_STRIPE
    _zero_stripe(rows0, agg_sh, base_row)
    plsc.subcore_barrier()
    w = c * NS + s
    h = NCH_E // 2
    for phase in range(2):
        pltpu.sync_copy(srcI_hbm.at[pl.ds(w * NCH_E + phase * h, h)], src_v)
        pltpu.sync_copy(dstI_hbm.at[pl.ds(w * NCH_E + phase * h, h)], dst_v)
        _gather_scatter_phase(table_hbm, src_v, dst_v,
                              (rows0, rows1, rows2, rows3), agg_sh,
                              (sg0, sg1, sg2, sg3), (ss0, ss1, ss2, ss3), h)
    plsc.subcore_barrier()
    _copy_out_stripe(rows0, agg_sh, out_hbm, base_row, c * NP + base_row)


_sc_prop_edge = pl.kernel(
    _prop_edge_body,
    out_type=jax.ShapeDtypeStruct((2 * NP, 128), jnp.float32),
    mesh=_MESH,
    scratch_types=[
        pltpu.VMEM((NCH_E // 2, B), jnp.int32),
        pltpu.VMEM((NCH_E // 2, B), jnp.int32),
        pltpu.VMEM((B, 128), jnp.float32),
        pltpu.VMEM((B, 128), jnp.float32),
        pltpu.VMEM((B, 128), jnp.float32),
        pltpu.VMEM((B, 128), jnp.float32),
        pltpu.VMEM_SHARED((ACC, 128), jnp.float32),
        pltpu.SemaphoreType.DMA,
        pltpu.SemaphoreType.DMA,
        pltpu.SemaphoreType.DMA,
        pltpu.SemaphoreType.DMA,
        pltpu.SemaphoreType.DMA,
        pltpu.SemaphoreType.DMA,
        pltpu.SemaphoreType.DMA,
        pltpu.SemaphoreType.DMA,
    ],
)


def _dis_from(d0_ref, d1_ref):
    deg = d0_ref[...] + d1_ref[...]
    return jnp.where(deg > 0, lax.rsqrt(jnp.maximum(deg, 1.0)), 0.0)


def _tc_in_body(x_ref, w_ref, d0_ref, d1_ref, out_ref):
    dis = _dis_from(d0_ref, d1_ref)
    h = jnp.dot(x_ref[...], w_ref[...], preferred_element_type=jnp.float32)
    hs = h * dis
    out_ref[0] = hs[:, :128]
    out_ref[1] = hs[:, 128:]


_tc_in = pl.pallas_call(
    _tc_in_body,
    grid=(GRID,),
    in_specs=[
        pl.BlockSpec((ROWB, D_IN), lambda i: (i, 0)),
        pl.BlockSpec((D_IN, D_H), lambda i: (0, 0)),
        pl.BlockSpec((ROWB, 1), lambda i: (i, 0)),
        pl.BlockSpec((ROWB, 1), lambda i: (i, 0)),
    ],
    out_specs=pl.BlockSpec((2, ROWB, 128), lambda i: (0, i, 0)),
    out_shape=jax.ShapeDtypeStruct((2, NP, 128), jnp.float32),
)


def _tc_mid_body(agg_ref, d0_ref, d1_ref, b_ref, w_ref, out_ref):
    dis = _dis_from(d0_ref, d1_ref)
    agg = jnp.concatenate([agg_ref[0], agg_ref[1]], axis=-1)
    t = jnp.maximum(agg * dis + b_ref[0], 0.0)
    h = jnp.dot(t, w_ref[...], preferred_element_type=jnp.float32)
    hs = h * dis
    out_ref[0] = hs[:, :128]
    out_ref[1] = hs[:, 128:]


_tc_mid = pl.pallas_call(
    _tc_mid_body,
    grid=(GRID,),
    in_specs=[
        pl.BlockSpec((2, ROWB, 128), lambda i: (0, i, 0)),
        pl.BlockSpec((ROWB, 1), lambda i: (i, 0)),
        pl.BlockSpec((ROWB, 1), lambda i: (i, 0)),
        pl.BlockSpec((1, D_H), lambda i: (0, 0)),
        pl.BlockSpec((D_H, D_H), lambda i: (0, 0)),
    ],
    out_specs=pl.BlockSpec((2, ROWB, 128), lambda i: (0, i, 0)),
    out_shape=jax.ShapeDtypeStruct((2, NP, 128), jnp.float32),
)


def _tc_out_body(agg_ref, d0_ref, d1_ref, b_ref, w_ref, out_ref):
    dis = _dis_from(d0_ref, d1_ref)
    agg = jnp.concatenate([agg_ref[0], agg_ref[1]], axis=-1)
    t = jnp.maximum(agg * dis + b_ref[0], 0.0)
    h = jnp.dot(t, w_ref[...], preferred_element_type=jnp.float32)
    hs = h * dis
    out_ref[...] = jnp.concatenate(
        [hs, jnp.zeros((ROWB, 128 - D_OUT), jnp.float32)], axis=-1)


_tc_out = pl.pallas_call(
    _tc_out_body,
    grid=(GRID,),
    in_specs=[
        pl.BlockSpec((2, ROWB, 128), lambda i: (0, i, 0)),
        pl.BlockSpec((ROWB, 1), lambda i: (i, 0)),
        pl.BlockSpec((ROWB, 1), lambda i: (i, 0)),
        pl.BlockSpec((1, D_H), lambda i: (0, 0)),
        pl.BlockSpec((D_H, D_OUT), lambda i: (0, 0)),
    ],
    out_specs=pl.BlockSpec((ROWB, 128), lambda i: (i, 0)),
    out_shape=jax.ShapeDtypeStruct((NP, 128), jnp.float32),
)


def _tc_final_body(aggp_ref, d0_ref, d1_ref, b_ref, out_ref):
    dis = _dis_from(d0_ref, d1_ref)
    p = aggp_ref[0, :, :D_OUT] + aggp_ref[1, :, :D_OUT]
    out_ref[...] = p * dis + b_ref[0]


_tc_final = pl.pallas_call(
    _tc_final_body,
    grid=(GRID,),
    in_specs=[
        pl.BlockSpec((2, ROWB, 128), lambda i: (0, i, 0)),
        pl.BlockSpec((ROWB, 1), lambda i: (i, 0)),
        pl.BlockSpec((ROWB, 1), lambda i: (i, 0)),
        pl.BlockSpec((1, D_OUT), lambda i: (0, 0)),
    ],
    out_specs=pl.BlockSpec((ROWB, D_OUT), lambda i: (i, 0)),
    out_shape=jax.ShapeDtypeStruct((N, D_OUT), jnp.float32),
)


def _pad_edges(x, per, pad_to, fill):
    groups = x.reshape(-1, per)
    padded = jnp.concatenate(
        [groups, jnp.broadcast_to(fill, (groups.shape[0], pad_to - per))],
        axis=1)
    return padded


def kernel(features, edge_index, W1, b1, W2, b2, W3, b3):
    src = edge_index[0].astype(jnp.int32)
    dst = edge_index[1].astype(jnp.int32)

    pad_c = N + (jnp.arange(EPT_PAD - EPT, dtype=jnp.int32) % (ACC - N))
    pad_e = N + (jnp.arange(EPW_PAD - EPW, dtype=jnp.int32) % (ACC - N))

    srcI_c = _pad_edges(src, EPT, EPT_PAD, jnp.int32(0)).reshape(-1, B)
    dstI_c = _pad_edges(dst, EPT, EPT_PAD, pad_c).reshape(-1, B)

    srcI_e = _pad_edges(src, EPW, EPW_PAD, jnp.int32(0)).reshape(-1, B)
    dstI_e = _pad_edges(dst, EPW, EPW_PAD, pad_e).reshape(-1, B)

    degp = _sc_deg(dstI_e)
    d0 = degp[:NP, :1]
    d1 = degp[NP:, :1]

    hs1 = _tc_in(features, W1, d0, d1)
    agg1 = _sc_prop_col(hs1.reshape(2 * NP, 128), srcI_c, dstI_c)
    hs2 = _tc_mid(agg1.reshape(2, NP, 128), d0, d1, b1.reshape(1, D_H), W2)
    agg2 = _sc_prop_col(hs2.reshape(2 * NP, 128), srcI_c, dstI_c)
    hs3 = _tc_out(agg2.reshape(2, NP, 128), d0, d1, b2.reshape(1, D_H), W3)
    agg3 = _sc_prop_edge(hs3, srcI_e, dstI_e)
    out = _tc_final(agg3.reshape(2, NP, 128), d0, d1, b3.reshape(1, D_OUT))
    return out


# speedup vs baseline: 4.6429x; 1.0147x over previous
import jax
import jax.numpy as jnp
from jax import lax
from jax.experimental import pallas as pl
from jax.experimental.pallas import tpu as pltpu
from jax.experimental.pallas import tpu_sc as plsc

N = 10000
E = 160000
D_IN = 256
D_H = 256
D_OUT = 64

NC = 2
NS = 16
NP = 12800
STRIPE = NP // NS
ROWB = 400
GRID = N // ROWB

ACC = 10240
ACC_STRIPE = ACC // NS
B = 128
EPT = E // NS
EPT_PAD = 10240
NCH_C = EPT_PAD // B
EPW = E // (NC * NS)
EPW_PAD = 5120
NCH_E = EPW_PAD // B

_MESH = plsc.VectorSubcoreMesh(core_axis_name="c", subcore_axis_name="s")


def _fill_const_rows(ref, nrows, ncols, value):
    v = jnp.full((16,), value, jnp.float32)

    def body(r, carry):
        for k in range(ncols // 16):
            ref[r, pl.ds(16 * k, 16)] = v
        return carry

    lax.fori_loop(0, nrows, body, 0)


def _zero_stripe(stage, agg_sh, base_row):
    _fill_const_rows(stage, B, 128, 0.0)
    for m in range(ACC_STRIPE // B):
        pltpu.sync_copy(stage, agg_sh.at[pl.ds(base_row + m * B, B)])


def _copy_out_stripe(stage, agg_sh, out_hbm, base_row, out_base):
    for m in range(ACC_STRIPE // B):
        pltpu.sync_copy(agg_sh.at[pl.ds(base_row + m * B, B)], stage)
        pltpu.sync_copy(stage, out_hbm.at[pl.ds(out_base + m * B, B)])


def _gather_scatter_phase(table_hbm, src_v, dst_v, rows, agg_sh,
                          sems_g, sems_s, nch):
    for b in range(2):
        pltpu.async_copy(table_hbm.at[src_v.at[b]], rows[b], sems_g[b])

    def pair(i, carry):
        for b in range(2):
            j = 2 * i + b
            pltpu.make_async_copy(table_hbm.at[src_v.at[j]], rows[b],
                                  sems_g[b]).wait()
            pltpu.async_copy(rows[b], agg_sh.at[dst_v.at[j]], sems_s[b],
                             add=True)
        for b in range(2):
            j = 2 * i + b
            pltpu.make_async_copy(rows[b], agg_sh.at[dst_v.at[j]],
                                  sems_s[b]).wait()
            pltpu.async_copy(table_hbm.at[src_v.at[j + 2]], rows[b],
                             sems_g[b])
        return carry

    lax.fori_loop(0, nch // 2 - 1, pair, 0)
    for b in range(2):
        j = nch - 2 + b
        pltpu.make_async_copy(table_hbm.at[src_v.at[j]], rows[b],
                              sems_g[b]).wait()
        pltpu.async_copy(rows[b], agg_sh.at[dst_v.at[j]], sems_s[b], add=True)
    for b in range(2):
        j = nch - 2 + b
        pltpu.make_async_copy(rows[b], agg_sh.at[dst_v.at[j]],
                              sems_s[b]).wait()


def _deg_body(dst_hbm, out_hbm, dst_v, ones_v, agg_sh, sem):
    c = lax.axis_index("c")
    s = lax.axis_index("s")
    base_row = s * ACC_STRIPE
    _zero_stripe(ones_v, agg_sh, base_row)
    plsc.subcore_barrier()
    _fill_const_rows(ones_v, B, 128, 1.0)
    w = c * NS + s
    pltpu.sync_copy(dst_hbm.at[pl.ds(w * NCH_E, NCH_E)], dst_v)

    def quad(i, carry):
        for b in range(4):
            j = 4 * i + b
            pltpu.async_copy(ones_v, agg_sh.at[dst_v.at[j]], sem, add=True)
        for b in range(4):
            j = 4 * i + b
            pltpu.make_async_copy(ones_v, agg_sh.at[dst_v.at[j]], sem).wait()
        return carry

    lax.fori_loop(0, NCH_E // 4, quad, 0)
    plsc.subcore_barrier()
    _copy_out_stripe(ones_v, agg_sh, out_hbm, base_row, c * NP + base_row)


_sc_deg = pl.kernel(
    _deg_body,
    out_type=jax.ShapeDtypeStruct((2 * NP, 128), jnp.float32),
    mesh=_MESH,
    scratch_types=[
        pltpu.VMEM((NCH_E, B), jnp.int32),
        pltpu.VMEM((B, 128), jnp.float32),
        pltpu.VMEM_SHARED((ACC, 128), jnp.float32),
        pltpu.SemaphoreType.DMA,
    ],
)


def _prop_col_body(table_hbm, srcI_hbm, dstI_hbm, out_hbm,
                   src_v, dst_v, rows0, rows1, agg_sh, sg0, sg1, ss0, ss1):
    c = lax.axis_index("c")
    s = lax.axis_index("s")
    base_row = s * ACC_STRIPE
    _zero_stripe(rows0, agg_sh, base_row)
    plsc.subcore_barrier()
    off = c * NP
    h = NCH_C // 2
    for phase in range(2):
        pltpu.sync_copy(srcI_hbm.at[pl.ds(s * NCH_C + phase * h, h)], src_v)
        pltpu.sync_copy(dstI_hbm.at[pl.ds(s * NCH_C + phase * h, h)], dst_v)

        def addoff(r, carry):
            for k in range(B // 16):
                src_v[r, pl.ds(16 * k, 16)] = src_v[r, pl.ds(16 * k, 16)] + off
            return carry

        lax.fori_loop(0, h, addoff, 0)
        _gather_scatter_phase(table_hbm, src_v, dst_v, (rows0, rows1), agg_sh,
                              (sg0, sg1), (ss0, ss1), h)
    plsc.subcore_barrier()
    _copy_out_stripe(rows0, agg_sh, out_hbm, base_row, c * NP + base_row)


_sc_prop_col = pl.kernel(
    _prop_col_body,
    out_type=jax.ShapeDtypeStruct((2 * NP, 128), jnp.float32),
    mesh=_MESH,
    scratch_types=[
        pltpu.VMEM((NCH_C // 2, B), jnp.int32),
        pltpu.VMEM((NCH_C // 2, B), jnp.int32),
        pltpu.VMEM((B, 128), jnp.float32),
        pltpu.VMEM((B, 128), jnp.float32),
        pltpu.VMEM_SHARED((ACC, 128), jnp.float32),
        pltpu.SemaphoreType.DMA,
        pltpu.SemaphoreType.DMA,
        pltpu.SemaphoreType.DMA,
        pltpu.SemaphoreType.DMA,
    ],
)


def _prop_edge_body(table_hbm, srcI_hbm, dstI_hbm, out_hbm,
                    src_v, dst_v, rows0, rows1, agg_sh, sg0, sg1, ss0, ss1):
    c = lax.axis_index("c")
    s = lax.axis_index("s")
    base_row = s * ACC_STRIPE
    _zero_stripe(rows0, agg_sh, base_row)
    plsc.subcore_barrier()
    w = c * NS + s
    h = NCH_E
    for phase in range(1):
        pltpu.sync_copy(srcI_hbm.at[pl.ds(w * NCH_E + phase * h, h)], src_v)
        pltpu.sync_copy(dstI_hbm.at[pl.ds(w * NCH_E + phase * h, h)], dst_v)
        _gather_scatter_phase(table_hbm, src_v, dst_v, (rows0, rows1),
                              agg_sh, (sg0, sg1), (ss0, ss1), h)
    plsc.subcore_barrier()
    _copy_out_stripe(rows0, agg_sh, out_hbm, base_row, c * NP + base_row)


_sc_prop_edge = pl.kernel(
    _prop_edge_body,
    out_type=jax.ShapeDtypeStruct((2 * NP, 128), jnp.float32),
    mesh=_MESH,
    scratch_types=[
        pltpu.VMEM((NCH_E, B), jnp.int32),
        pltpu.VMEM((NCH_E, B), jnp.int32),
        pltpu.VMEM((B, 128), jnp.float32),
        pltpu.VMEM((B, 128), jnp.float32),
        pltpu.VMEM_SHARED((ACC, 128), jnp.float32),
        pltpu.SemaphoreType.DMA,
        pltpu.SemaphoreType.DMA,
        pltpu.SemaphoreType.DMA,
        pltpu.SemaphoreType.DMA,
    ],
)


def _dis_from(d0_ref, d1_ref):
    deg = d0_ref[...] + d1_ref[...]
    return jnp.where(deg > 0, lax.rsqrt(jnp.maximum(deg, 1.0)), 0.0)


def _tc_in_body(x_ref, w_ref, d0_ref, d1_ref, out_ref):
    dis = _dis_from(d0_ref, d1_ref)
    h = jnp.dot(x_ref[...], w_ref[...], preferred_element_type=jnp.float32)
    hs = h * dis
    out_ref[0] = hs[:, :128]
    out_ref[1] = hs[:, 128:]


_tc_in = pl.pallas_call(
    _tc_in_body,
    grid=(GRID,),
    in_specs=[
        pl.BlockSpec((ROWB, D_IN), lambda i: (i, 0)),
        pl.BlockSpec((D_IN, D_H), lambda i: (0, 0)),
        pl.BlockSpec((ROWB, 1), lambda i: (i, 0)),
        pl.BlockSpec((ROWB, 1), lambda i: (i, 0)),
    ],
    out_specs=pl.BlockSpec((2, ROWB, 128), lambda i: (0, i, 0)),
    out_shape=jax.ShapeDtypeStruct((2, NP, 128), jnp.float32),
)


def _tc_mid_body(agg_ref, d0_ref, d1_ref, b_ref, w_ref, out_ref):
    dis = _dis_from(d0_ref, d1_ref)
    agg = jnp.concatenate([agg_ref[0], agg_ref[1]], axis=-1)
    t = jnp.maximum(agg * dis + b_ref[0], 0.0)
    h = jnp.dot(t, w_ref[...], preferred_element_type=jnp.float32)
    hs = h * dis
    out_ref[0] = hs[:, :128]
    out_ref[1] = hs[:, 128:]


_tc_mid = pl.pallas_call(
    _tc_mid_body,
    grid=(GRID,),
    in_specs=[
        pl.BlockSpec((2, ROWB, 128), lambda i: (0, i, 0)),
        pl.BlockSpec((ROWB, 1), lambda i: (i, 0)),
        pl.BlockSpec((ROWB, 1), lambda i: (i, 0)),
        pl.BlockSpec((1, D_H), lambda i: (0, 0)),
        pl.BlockSpec((D_H, D_H), lambda i: (0, 0)),
    ],
    out_specs=pl.BlockSpec((2, ROWB, 128), lambda i: (0, i, 0)),
    out_shape=jax.ShapeDtypeStruct((2, NP, 128), jnp.float32),
)


def _tc_out_body(agg_ref, d0_ref, d1_ref, b_ref, w_ref, out_ref):
    dis = _dis_from(d0_ref, d1_ref)
    agg = jnp.concatenate([agg_ref[0], agg_ref[1]], axis=-1)
    t = jnp.maximum(agg * dis + b_ref[0], 0.0)
    h = jnp.dot(t, w_ref[...], preferred_element_type=jnp.float32)
    hs = h * dis
    hsp = jnp.concatenate(
        [hs, jnp.zeros((ROWB, 128 - D_OUT), jnp.float32)], axis=-1)
    out_ref[0] = hsp
    out_ref[1] = hsp


_tc_out = pl.pallas_call(
    _tc_out_body,
    grid=(GRID,),
    in_specs=[
        pl.BlockSpec((2, ROWB, 128), lambda i: (0, i, 0)),
        pl.BlockSpec((ROWB, 1), lambda i: (i, 0)),
        pl.BlockSpec((ROWB, 1), lambda i: (i, 0)),
        pl.BlockSpec((1, D_H), lambda i: (0, 0)),
        pl.BlockSpec((D_H, D_OUT), lambda i: (0, 0)),
    ],
    out_specs=pl.BlockSpec((2, ROWB, 128), lambda i: (0, i, 0)),
    out_shape=jax.ShapeDtypeStruct((2, NP, 128), jnp.float32),
)


def _tc_final_body(aggp_ref, d0_ref, d1_ref, b_ref, out_ref):
    dis = _dis_from(d0_ref, d1_ref)
    p = aggp_ref[0, :, :D_OUT] + aggp_ref[1, :, :D_OUT]
    out_ref[...] = p * dis + b_ref[0]


_tc_final = pl.pallas_call(
    _tc_final_body,
    grid=(GRID,),
    in_specs=[
        pl.BlockSpec((2, ROWB, 128), lambda i: (0, i, 0)),
        pl.BlockSpec((ROWB, 1), lambda i: (i, 0)),
        pl.BlockSpec((ROWB, 1), lambda i: (i, 0)),
        pl.BlockSpec((1, D_OUT), lambda i: (0, 0)),
    ],
    out_specs=pl.BlockSpec((ROWB, D_OUT), lambda i: (i, 0)),
    out_shape=jax.ShapeDtypeStruct((N, D_OUT), jnp.float32),
)


def _pad_edges(x, per, pad_to, fill):
    groups = x.reshape(-1, per)
    padded = jnp.concatenate(
        [groups, jnp.broadcast_to(fill, (groups.shape[0], pad_to - per))],
        axis=1)
    return padded


def kernel(features, edge_index, W1, b1, W2, b2, W3, b3):
    src = edge_index[0].astype(jnp.int32)
    dst = edge_index[1].astype(jnp.int32)

    pad_c = N + (jnp.arange(EPT_PAD - EPT, dtype=jnp.int32) % (ACC - N))
    pad_e = N + (jnp.arange(EPW_PAD - EPW, dtype=jnp.int32) % (ACC - N))

    srcI_c = _pad_edges(src, EPT, EPT_PAD, jnp.int32(0)).reshape(-1, B)
    dstI_c = _pad_edges(dst, EPT, EPT_PAD, pad_c).reshape(-1, B)

    srcI_e = _pad_edges(src, EPW, EPW_PAD, jnp.int32(0))
    core_off = (jnp.arange(NC * NS, dtype=jnp.int32)[:, None] // NS) * NP
    srcI_e = (srcI_e + core_off).reshape(-1, B)
    dstI_e = _pad_edges(dst, EPW, EPW_PAD, pad_e).reshape(-1, B)

    degp = _sc_deg(dstI_e)
    d0 = degp[:NP, :1]
    d1 = degp[NP:, :1]

    hs1 = _tc_in(features, W1, d0, d1)
    agg1 = _sc_prop_col(hs1.reshape(2 * NP, 128), srcI_c, dstI_c)
    hs2 = _tc_mid(agg1.reshape(2, NP, 128), d0, d1, b1.reshape(1, D_H), W2)
    agg2 = _sc_prop_col(hs2.reshape(2 * NP, 128), srcI_c, dstI_c)
    hs3 = _tc_out(agg2.reshape(2, NP, 128), d0, d1, b2.reshape(1, D_H), W3)
    agg3 = _sc_prop_edge(hs3.reshape(2 * NP, 128), srcI_e, dstI_e)
    out = _tc_final(agg3.reshape(2, NP, 128), d0, d1, b3.reshape(1, D_OUT))
    return out
